# baseline (speedup 1.0000x reference)
import functools
import math

import jax
import jax.numpy as jnp
from jax.experimental import pallas as pl
from jax.experimental.pallas import tpu as pltpu


_TOP_K = 66
_NEG = -1e9


def _ln(x, g, b, eps=1e-5):
    mu = jnp.mean(x, axis=-1, keepdims=True)
    var = jnp.mean((x - mu) ** 2, axis=-1, keepdims=True)
    return (x - mu) * jax.lax.rsqrt(var + eps) * g + b


def _gelu(x):
    return 0.5 * x * (1.0 + jax.lax.erf(x * (1.0 / math.sqrt(2.0))))


def _encoder_body(x_ref, w1t_ref, b1_ref, g1_ref, be1_ref, w2t_ref, b2_ref,
                  g2_ref, be2_ref, wqt_ref, wkt_ref, h_ref, q_ref, k_ref):
    x = jnp.nan_to_num(x_ref[0], nan=0.0, posinf=0.0, neginf=0.0)
    h1 = jnp.dot(x, w1t_ref[...], preferred_element_type=jnp.float32)
    h1 = _gelu(_ln(h1 + b1_ref[...], g1_ref[...], be1_ref[...]))
    h2 = jnp.dot(h1, w2t_ref[...], preferred_element_type=jnp.float32)
    h2 = _gelu(_ln(h2 + b2_ref[...], g2_ref[...], be2_ref[...]))
    h_ref[0] = h2
    q_ref[0] = jnp.dot(h2, wqt_ref[...], preferred_element_type=jnp.float32)
    k_ref[0] = jnp.dot(h2, wkt_ref[...], preferred_element_type=jnp.float32)


def _adj_body(q_ref, k_ref, adj_ref, stats_ref, *, rb, n, k_top):
    nb = pl.program_id(1)
    q = q_ref[0]
    kk = k_ref[0]
    inv = 1.0 / math.sqrt(q.shape[-1])
    row0 = nb * rb

    st = jax.lax.dot_general(kk, q, (((1,), (1,)), ((), ())),
                             preferred_element_type=jnp.float32) * inv
    cols_t = jax.lax.broadcasted_iota(jnp.int32, (n, rb), 0)
    rows_t = row0 + jax.lax.broadcasted_iota(jnp.int32, (n, rb), 1)
    st = jnp.where(cols_t == rows_t, _NEG, st)
    wt = jax.nn.sigmoid(st)

    wb = wt.astype(jnp.bfloat16)
    lo0 = jnp.zeros((1, rb), jnp.int32)
    hi0 = jnp.full((1, rb), 0x3F81, jnp.int32)

    def body16(_, carry):
        lo, hi = carry
        mid = lo + ((hi - lo) >> 1)
        t16 = jax.lax.bitcast_convert_type(mid.astype(jnp.int16),
                                           jnp.bfloat16)
        cnt = jnp.count_nonzero(wb >= t16, axis=0, keepdims=True)
        ge = cnt >= k_top
        return jnp.where(ge, mid, lo), jnp.where(ge, hi, mid)

    lo16, _ = jax.lax.fori_loop(0, 14, body16, (lo0, hi0))
    base = lo16 << 16
    lo320 = jnp.maximum(base - 0x8001, 0)
    hi320 = base + 0x8001

    def body(_, carry):
        lo, hi = carry
        mid = lo + ((hi - lo) >> 1)
        t = jax.lax.bitcast_convert_type(mid, jnp.float32)
        cnt = jnp.count_nonzero(wt >= t, axis=0, keepdims=True)
        ge = cnt >= k_top
        return jnp.where(ge, mid, lo), jnp.where(ge, hi, mid)

    lo, _ = jax.lax.fori_loop(0, 17, body, (lo320, hi320))
    vk = jax.lax.bitcast_convert_type(lo, jnp.float32)

    adjt = jnp.where(wt >= vk, wt, 0.0)
    deg = jnp.sum((adjt > 0.0).astype(jnp.float32), axis=0, keepdims=True)
    rs = jnp.sum(adjt, axis=0, keepdims=True)
    rmax = jnp.max(wt, axis=0, keepdims=True)
    r_row = rs + 1e-8
    c_row = 1e-8 * r_row
    alog = jnp.sum(adjt * jnp.log(adjt + c_row), axis=0, keepdims=True)
    ent = (jnp.log(r_row) * rs - alog) / r_row
    stats_ref[0, 0, :] = deg[0]
    stats_ref[0, 1, :] = rs[0]
    stats_ref[0, 2, :] = rmax[0]
    stats_ref[0, 3, :] = ent[0]

    adj_ref[0] = jnp.transpose(adjt)


def _head_body(stats_ref, mf_ref, cw1t_ref, cb1_ref, clg_ref, clb_ref,
               cw2t_ref, cb2_ref, tw1t_ref, tb1_ref, tw2t_ref, tb2_ref,
               props_ref, logits_ref, probs_ref, regime_ref, conf_ref,
               tp_ref, *, n, npad):
    st = stats_ref[...]
    bsz = st.shape[0]
    valid = jax.lax.broadcasted_iota(jnp.int32, (bsz, npad), 1) < n
    deg = jnp.where(valid, st[:, 0, :], 0.0)
    rs = jnp.where(valid, st[:, 1, :], 0.0)
    rmax = jnp.where(valid, st[:, 2, :], 0.0)
    ent = jnp.where(valid, st[:, 3, :], 0.0)

    edge_count = jnp.sum(deg, axis=1)
    possible = float(max(n * (n - 1), 1))
    density = edge_count / possible
    mean_deg = edge_count / float(n)
    mean_degree_norm = mean_deg / float(max(n - 1, 1))
    dev = jnp.where(valid, deg - mean_deg[:, None], 0.0)
    var_deg = jnp.sum(dev * dev, axis=1) / float(n - 1)
    std_degree_norm = jnp.sqrt(var_deg) / float(max(n - 1, 1))
    weight_sum = jnp.sum(rs, axis=1)
    mean_weight = weight_sum / (edge_count + 1e-8)
    max_weight = jnp.max(rmax, axis=1)
    entropy = (jnp.sum(ent, axis=1) / float(n)) / math.log(max(n, 2))
    graph_stress = 0.5 * density + 0.5 * mean_weight

    props = jnp.stack([density, mean_degree_norm, std_degree_norm,
                       mean_weight, max_weight, entropy, graph_stress],
                      axis=1)
    props_ref[...] = props
    mf = jnp.nan_to_num(mf_ref[...], nan=0.0, posinf=0.0, neginf=0.0)
    clf_in = jnp.concatenate([props, mf], axis=1)

    hc = jnp.dot(clf_in, cw1t_ref[...], preferred_element_type=jnp.float32)
    hc = _gelu(_ln(hc + cb1_ref[...], clg_ref[...], clb_ref[...]))
    logits = jnp.dot(hc, cw2t_ref[...],
                     preferred_element_type=jnp.float32) + cb2_ref[...]
    logits_ref[...] = logits
    m = jnp.max(logits, axis=1, keepdims=True)
    e = jnp.exp(logits - m)
    probs = e / jnp.sum(e, axis=1, keepdims=True)
    probs_ref[...] = probs

    maxv = jnp.max(probs, axis=1, keepdims=True)
    idx = jax.lax.broadcasted_iota(jnp.int32, probs.shape, 1)
    cand = jnp.where(probs == maxv, idx, jnp.int32(2**30))
    regime_ref[...] = jnp.min(cand, axis=1)[None, :]
    conf_ref[...] = maxv[:, 0][None, :]

    ht = jnp.dot(clf_in, tw1t_ref[...], preferred_element_type=jnp.float32)
    ht = _gelu(ht + tb1_ref[...])
    tp = jax.nn.sigmoid(
        jnp.dot(ht, tw2t_ref[...],
                preferred_element_type=jnp.float32) + tb2_ref[...])
    tp_ref[...] = tp[:, 0][None, :]


def kernel(node_features, macro_features, enc_w1, enc_b1, ln1_g, ln1_b,
           enc_w2, enc_b2, ln2_g, ln2_b, Wq, Wk, clf_w1, clf_b1, clf_ln_g,
           clf_ln_b, clf_w2, clf_b2, tr_w1, tr_b1, tr_w2, tr_b2):
    x = node_features.astype(jnp.float32)
    bsz, n, d = x.shape
    hdim = enc_w1.shape[0]
    g = enc_w2.shape[0]
    k_top = min(max(1, _TOP_K), max(1, n - 1))

    rbe = 512
    nbe = pl.cdiv(n, rbe)
    enc = pl.pallas_call(
        _encoder_body,
        grid=(bsz, nbe),
        in_specs=[
            pl.BlockSpec((1, rbe, d), lambda b, i: (b, i, 0)),
            pl.BlockSpec((d, hdim), lambda b, i: (0, 0)),
            pl.BlockSpec((1, hdim), lambda b, i: (0, 0)),
            pl.BlockSpec((1, hdim), lambda b, i: (0, 0)),
            pl.BlockSpec((1, hdim), lambda b, i: (0, 0)),
            pl.BlockSpec((hdim, g), lambda b, i: (0, 0)),
            pl.BlockSpec((1, g), lambda b, i: (0, 0)),
            pl.BlockSpec((1, g), lambda b, i: (0, 0)),
            pl.BlockSpec((1, g), lambda b, i: (0, 0)),
            pl.BlockSpec((g, g), lambda b, i: (0, 0)),
            pl.BlockSpec((g, g), lambda b, i: (0, 0)),
        ],
        out_specs=[
            pl.BlockSpec((1, rbe, g), lambda b, i: (b, i, 0)),
            pl.BlockSpec((1, rbe, g), lambda b, i: (b, i, 0)),
            pl.BlockSpec((1, rbe, g), lambda b, i: (b, i, 0)),
        ],
        out_shape=[jax.ShapeDtypeStruct((bsz, n, g), jnp.float32)] * 3,
    )(x, enc_w1.T, enc_b1[None, :], ln1_g[None, :], ln1_b[None, :],
      enc_w2.T, enc_b2[None, :], ln2_g[None, :], ln2_b[None, :],
      Wq.T, Wk.T)
    h, q, k = enc

    rb = 640
    nb = pl.cdiv(n, rb)
    npad = nb * rb
    adj, stats = pl.pallas_call(
        functools.partial(_adj_body, rb=rb, n=n, k_top=k_top),
        grid=(bsz, nb),
        in_specs=[
            pl.BlockSpec((1, rb, g), lambda b, i: (b, i, 0)),
            pl.BlockSpec((1, n, g), lambda b, i: (b, 0, 0)),
        ],
        out_specs=[
            pl.BlockSpec((1, rb, n), lambda b, i: (b, i, 0)),
            pl.BlockSpec((1, 4, rb), lambda b, i: (b, 0, i)),
        ],
        out_shape=[
            jax.ShapeDtypeStruct((bsz, n, n), jnp.float32),
            jax.ShapeDtypeStruct((bsz, 4, npad), jnp.float32),
        ],
    )(q, k)

    props, logits, probs, regime, conf, tp = pl.pallas_call(
        functools.partial(_head_body, n=n, npad=npad),
        in_specs=[pl.BlockSpec(stats.shape, lambda: (0, 0, 0)),
                  pl.BlockSpec(macro_features.shape, lambda: (0, 0))] +
                 [pl.BlockSpec(s, lambda: (0, 0)) for s in [
                     clf_w1.T.shape, (1, clf_b1.shape[0]),
                     (1, clf_ln_g.shape[0]), (1, clf_ln_b.shape[0]),
                     clf_w2.T.shape, (1, clf_b2.shape[0]),
                     tr_w1.T.shape, (1, tr_b1.shape[0]),
                     tr_w2.T.shape, (1, tr_b2.shape[0])]],
        out_specs=[
            pl.BlockSpec((bsz, 7), lambda: (0, 0)),
            pl.BlockSpec((bsz, 4), lambda: (0, 0)),
            pl.BlockSpec((bsz, 4), lambda: (0, 0)),
            pl.BlockSpec((1, bsz), lambda: (0, 0)),
            pl.BlockSpec((1, bsz), lambda: (0, 0)),
            pl.BlockSpec((1, bsz), lambda: (0, 0)),
        ],
        out_shape=[
            jax.ShapeDtypeStruct((bsz, 7), jnp.float32),
            jax.ShapeDtypeStruct((bsz, 4), jnp.float32),
            jax.ShapeDtypeStruct((bsz, 4), jnp.float32),
            jax.ShapeDtypeStruct((1, bsz), jnp.int32),
            jax.ShapeDtypeStruct((1, bsz), jnp.float32),
            jax.ShapeDtypeStruct((1, bsz), jnp.float32),
        ],
    )(stats, macro_features.astype(jnp.float32),
      clf_w1.T, clf_b1[None, :], clf_ln_g[None, :], clf_ln_b[None, :],
      clf_w2.T, clf_b2[None, :],
      tr_w1.T, tr_b1[None, :], tr_w2.T, tr_b2[None, :])

    return (logits, probs, regime[0], conf[0], tp[0], adj, props, h)


# speedup vs baseline: 1.6293x; 1.6293x over previous
import functools
import math

import jax
import jax.numpy as jnp
from jax.experimental import pallas as pl
from jax.experimental.pallas import tpu as pltpu


_TOP_K = 66
_NEG = -1e9


def _ln(x, g, b, eps=1e-5):
    mu = jnp.mean(x, axis=-1, keepdims=True)
    var = jnp.mean((x - mu) ** 2, axis=-1, keepdims=True)
    return (x - mu) * jax.lax.rsqrt(var + eps) * g + b


def _gelu(x):
    return 0.5 * x * (1.0 + jax.lax.erf(x * (1.0 / math.sqrt(2.0))))


def _encoder_body(x_ref, w1t_ref, b1_ref, g1_ref, be1_ref, w2t_ref, b2_ref,
                  g2_ref, be2_ref, wqt_ref, wkt_ref, h_ref, q_ref, k_ref):
    x = jnp.nan_to_num(x_ref[0], nan=0.0, posinf=0.0, neginf=0.0)
    h1 = jnp.dot(x, w1t_ref[...], preferred_element_type=jnp.float32)
    h1 = _gelu(_ln(h1 + b1_ref[...], g1_ref[...], be1_ref[...]))
    h2 = jnp.dot(h1, w2t_ref[...], preferred_element_type=jnp.float32)
    h2 = _gelu(_ln(h2 + b2_ref[...], g2_ref[...], be2_ref[...]))
    h_ref[0] = h2
    q_ref[0] = jnp.dot(h2, wqt_ref[...], preferred_element_type=jnp.float32)
    k_ref[0] = jnp.dot(h2, wkt_ref[...], preferred_element_type=jnp.float32)


def _adj_body(q_ref, k_ref, adj_ref, stats_ref, *, rb, n, k_top):
    nb = pl.program_id(1)
    q = q_ref[0]
    kk = k_ref[0]
    inv = 1.0 / math.sqrt(q.shape[-1])
    row0 = nb * rb

    st = jax.lax.dot_general(kk, q, (((1,), (1,)), ((), ())),
                             preferred_element_type=jnp.float32) * inv
    cols_t = jax.lax.broadcasted_iota(jnp.int32, (n, rb), 0)
    rows_t = row0 + jax.lax.broadcasted_iota(jnp.int32, (n, rb), 1)
    st = jnp.where(cols_t == rows_t, _NEG, st)
    wt = jax.nn.sigmoid(st)

    lo0 = jnp.zeros((1, rb), jnp.int32)
    hi0 = jnp.full((1, rb), 0x3F800001, jnp.int32)

    def body(_, carry):
        lo, hi = carry
        mid = lo + ((hi - lo) >> 1)
        t = jax.lax.bitcast_convert_type(mid, jnp.float32)
        cnt = jnp.count_nonzero(wt >= t, axis=0, keepdims=True)
        ge = cnt >= k_top
        return jnp.where(ge, mid, lo), jnp.where(ge, hi, mid)

    lo, _ = jax.lax.fori_loop(0, 30, body, (lo0, hi0))
    vk = jax.lax.bitcast_convert_type(lo, jnp.float32)

    adjt = jnp.where(wt >= vk, wt, 0.0)
    deg = jnp.sum((adjt > 0.0).astype(jnp.float32), axis=0, keepdims=True)
    rs = jnp.sum(adjt, axis=0, keepdims=True)
    rmax = jnp.max(wt, axis=0, keepdims=True)
    r_row = rs + 1e-8
    c_row = 1e-8 * r_row
    alog = jnp.sum(adjt * jnp.log(adjt + c_row), axis=0, keepdims=True)
    ent = (jnp.log(r_row) * rs - alog) / r_row
    stats_ref[0, 0, :] = deg[0]
    stats_ref[0, 1, :] = rs[0]
    stats_ref[0, 2, :] = rmax[0]
    stats_ref[0, 3, :] = ent[0]

    adj_ref[0] = jnp.transpose(adjt)


def _head_body(stats_ref, mf_ref, cw1t_ref, cb1_ref, clg_ref, clb_ref,
               cw2t_ref, cb2_ref, tw1t_ref, tb1_ref, tw2t_ref, tb2_ref,
               props_ref, logits_ref, probs_ref, regime_ref, conf_ref,
               tp_ref, *, n, npad):
    st = stats_ref[...]
    bsz = st.shape[0]
    valid = jax.lax.broadcasted_iota(jnp.int32, (bsz, npad), 1) < n
    deg = jnp.where(valid, st[:, 0, :], 0.0)
    rs = jnp.where(valid, st[:, 1, :], 0.0)
    rmax = jnp.where(valid, st[:, 2, :], 0.0)
    ent = jnp.where(valid, st[:, 3, :], 0.0)

    edge_count = jnp.sum(deg, axis=1)
    possible = float(max(n * (n - 1), 1))
    density = edge_count / possible
    mean_deg = edge_count / float(n)
    mean_degree_norm = mean_deg / float(max(n - 1, 1))
    dev = jnp.where(valid, deg - mean_deg[:, None], 0.0)
    var_deg = jnp.sum(dev * dev, axis=1) / float(n - 1)
    std_degree_norm = jnp.sqrt(var_deg) / float(max(n - 1, 1))
    weight_sum = jnp.sum(rs, axis=1)
    mean_weight = weight_sum / (edge_count + 1e-8)
    max_weight = jnp.max(rmax, axis=1)
    entropy = (jnp.sum(ent, axis=1) / float(n)) / math.log(max(n, 2))
    graph_stress = 0.5 * density + 0.5 * mean_weight

    props = jnp.stack([density, mean_degree_norm, std_degree_norm,
                       mean_weight, max_weight, entropy, graph_stress],
                      axis=1)
    props_ref[...] = props
    mf = jnp.nan_to_num(mf_ref[...], nan=0.0, posinf=0.0, neginf=0.0)
    clf_in = jnp.concatenate([props, mf], axis=1)

    hc = jnp.dot(clf_in, cw1t_ref[...], preferred_element_type=jnp.float32)
    hc = _gelu(_ln(hc + cb1_ref[...], clg_ref[...], clb_ref[...]))
    logits = jnp.dot(hc, cw2t_ref[...],
                     preferred_element_type=jnp.float32) + cb2_ref[...]
    logits_ref[...] = logits
    m = jnp.max(logits, axis=1, keepdims=True)
    e = jnp.exp(logits - m)
    probs = e / jnp.sum(e, axis=1, keepdims=True)
    probs_ref[...] = probs

    maxv = jnp.max(probs, axis=1, keepdims=True)
    idx = jax.lax.broadcasted_iota(jnp.int32, probs.shape, 1)
    cand = jnp.where(probs == maxv, idx, jnp.int32(2**30))
    regime_ref[...] = jnp.min(cand, axis=1)[None, :]
    conf_ref[...] = maxv[:, 0][None, :]

    ht = jnp.dot(clf_in, tw1t_ref[...], preferred_element_type=jnp.float32)
    ht = _gelu(ht + tb1_ref[...])
    tp = jax.nn.sigmoid(
        jnp.dot(ht, tw2t_ref[...],
                preferred_element_type=jnp.float32) + tb2_ref[...])
    tp_ref[...] = tp[:, 0][None, :]


def kernel(node_features, macro_features, enc_w1, enc_b1, ln1_g, ln1_b,
           enc_w2, enc_b2, ln2_g, ln2_b, Wq, Wk, clf_w1, clf_b1, clf_ln_g,
           clf_ln_b, clf_w2, clf_b2, tr_w1, tr_b1, tr_w2, tr_b2):
    x = node_features.astype(jnp.float32)
    bsz, n, d = x.shape
    hdim = enc_w1.shape[0]
    g = enc_w2.shape[0]
    k_top = min(max(1, _TOP_K), max(1, n - 1))

    rbe = 512
    nbe = pl.cdiv(n, rbe)
    enc = pl.pallas_call(
        _encoder_body,
        grid=(bsz, nbe),
        in_specs=[
            pl.BlockSpec((1, rbe, d), lambda b, i: (b, i, 0)),
            pl.BlockSpec((d, hdim), lambda b, i: (0, 0)),
            pl.BlockSpec((1, hdim), lambda b, i: (0, 0)),
            pl.BlockSpec((1, hdim), lambda b, i: (0, 0)),
            pl.BlockSpec((1, hdim), lambda b, i: (0, 0)),
            pl.BlockSpec((hdim, g), lambda b, i: (0, 0)),
            pl.BlockSpec((1, g), lambda b, i: (0, 0)),
            pl.BlockSpec((1, g), lambda b, i: (0, 0)),
            pl.BlockSpec((1, g), lambda b, i: (0, 0)),
            pl.BlockSpec((g, g), lambda b, i: (0, 0)),
            pl.BlockSpec((g, g), lambda b, i: (0, 0)),
        ],
        out_specs=[
            pl.BlockSpec((1, rbe, g), lambda b, i: (b, i, 0)),
            pl.BlockSpec((1, rbe, g), lambda b, i: (b, i, 0)),
            pl.BlockSpec((1, rbe, g), lambda b, i: (b, i, 0)),
        ],
        out_shape=[jax.ShapeDtypeStruct((bsz, n, g), jnp.float32)] * 3,
    )(x, enc_w1.T, enc_b1[None, :], ln1_g[None, :], ln1_b[None, :],
      enc_w2.T, enc_b2[None, :], ln2_g[None, :], ln2_b[None, :],
      Wq.T, Wk.T)
    h, q, k = enc

    rb = 1280
    nb = pl.cdiv(n, rb)
    npad = nb * rb
    adj, stats = pl.pallas_call(
        functools.partial(_adj_body, rb=rb, n=n, k_top=k_top),
        grid=(bsz, nb),
        in_specs=[
            pl.BlockSpec((1, rb, g), lambda b, i: (b, i, 0)),
            pl.BlockSpec((1, n, g), lambda b, i: (b, 0, 0)),
        ],
        out_specs=[
            pl.BlockSpec((1, rb, n), lambda b, i: (b, i, 0)),
            pl.BlockSpec((1, 4, rb), lambda b, i: (b, 0, i)),
        ],
        out_shape=[
            jax.ShapeDtypeStruct((bsz, n, n), jnp.float32),
            jax.ShapeDtypeStruct((bsz, 4, npad), jnp.float32),
        ],
    )(q, k)

    props, logits, probs, regime, conf, tp = pl.pallas_call(
        functools.partial(_head_body, n=n, npad=npad),
        in_specs=[pl.BlockSpec(stats.shape, lambda: (0, 0, 0)),
                  pl.BlockSpec(macro_features.shape, lambda: (0, 0))] +
                 [pl.BlockSpec(s, lambda: (0, 0)) for s in [
                     clf_w1.T.shape, (1, clf_b1.shape[0]),
                     (1, clf_ln_g.shape[0]), (1, clf_ln_b.shape[0]),
                     clf_w2.T.shape, (1, clf_b2.shape[0]),
                     tr_w1.T.shape, (1, tr_b1.shape[0]),
                     tr_w2.T.shape, (1, tr_b2.shape[0])]],
        out_specs=[
            pl.BlockSpec((bsz, 7), lambda: (0, 0)),
            pl.BlockSpec((bsz, 4), lambda: (0, 0)),
            pl.BlockSpec((bsz, 4), lambda: (0, 0)),
            pl.BlockSpec((1, bsz), lambda: (0, 0)),
            pl.BlockSpec((1, bsz), lambda: (0, 0)),
            pl.BlockSpec((1, bsz), lambda: (0, 0)),
        ],
        out_shape=[
            jax.ShapeDtypeStruct((bsz, 7), jnp.float32),
            jax.ShapeDtypeStruct((bsz, 4), jnp.float32),
            jax.ShapeDtypeStruct((bsz, 4), jnp.float32),
            jax.ShapeDtypeStruct((1, bsz), jnp.int32),
            jax.ShapeDtypeStruct((1, bsz), jnp.float32),
            jax.ShapeDtypeStruct((1, bsz), jnp.float32),
        ],
    )(stats, macro_features.astype(jnp.float32),
      clf_w1.T, clf_b1[None, :], clf_ln_g[None, :], clf_ln_b[None, :],
      clf_w2.T, clf_b2[None, :],
      tr_w1.T, tr_b1[None, :], tr_w2.T, tr_b2[None, :])

    return (logits, probs, regime[0], conf[0], tp[0], adj, props, h)
